# baseline (speedup 1.0000x reference)
import functools
import math

import jax
import jax.numpy as jnp
from jax import lax
from jax.experimental import pallas as pl
from jax.experimental.pallas import tpu as pltpu
from jax.experimental.pallas import tpu_sc as plsc

_M = 0.1
_LN = math.log(1.0 - _M)

_B = 4096
_F = 128
_RB = 1024
_NB = _B // _RB


def _sc_gather(mem, targets):
    info = plsc.get_sparse_core_info()
    nw = info.num_cores * info.num_subcores
    bpw = _B // nw
    mesh = plsc.VectorSubcoreMesh(core_axis_name="c", subcore_axis_name="s")

    @functools.partial(
        pl.kernel,
        mesh=mesh,
        out_type=jax.ShapeDtypeStruct((_B, _F), jnp.float32),
        scratch_types=[
            pltpu.VMEM((bpw,), jnp.int32),
            pltpu.VMEM((bpw, _F), jnp.float32),
            pltpu.SemaphoreType.DMA,
        ],
    )
    def k(table_hbm, idx_hbm, out_hbm, idx_v, rows_v, sem):
        wid = lax.axis_index("s") * info.num_cores + lax.axis_index("c")
        base = wid * bpw
        pltpu.sync_copy(idx_hbm.at[pl.ds(base, bpw)], idx_v)
        pltpu.async_copy(table_hbm.at[idx_v], rows_v, sem).wait()
        pltpu.sync_copy(rows_v, out_hbm.at[pl.ds(base, bpw)])

    return k(mem, targets)


_CMAX = 200.0


def _tc_body(t_sub_ref, t_lane_ref, r_ref, corr_ref, coef_ref, rp_scr,
             maxc_ref):
    i = pl.program_id(0)
    r0 = i * _RB
    t_row = t_sub_ref[pl.ds(r0, _RB), :]
    t_col_i = t_lane_ref[pl.ds(i, 1), :]
    row_iota = lax.broadcasted_iota(jnp.int32, (_RB, _RB), 0)
    col_iota = lax.broadcasted_iota(jnp.int32, (_RB, _RB), 1)

    eq_diag = t_row == t_col_i
    mask_strict = eq_diag & (col_iota < row_iota)
    ws = jnp.where(mask_strict, 1.0, 0.0)

    def ablk(l, acc):
        t_chunk = t_lane_ref[pl.ds(l, 1), :]
        w = jnp.where(t_row == t_chunk, 1.0, 0.0)
        return acc + lax.dot_general(
            w, rp_scr[pl.ds(l * _RB, _RB), :], (((1,), (0,)), ((), ())),
            preferred_element_type=jnp.float32)

    acc = lax.fori_loop(0, i, ablk, jnp.zeros((_RB, _F + 8), jnp.float32))
    cntd = lax.dot_general(
        ws, jnp.where(lax.broadcasted_iota(jnp.int32, (_RB, 8), 1) == 0,
                      1.0, 0.0),
        (((1,), (0,)), ((), ())), preferred_element_type=jnp.float32)
    c_row = acc[:, _F:_F + 1] + cntd[:, 0:1]

    mx = jnp.max(c_row)
    mxc = jnp.where(i == 0, mx, jnp.maximum(maxc_ref[0], mx))
    maxc_ref[0] = mxc
    ones_pad = jnp.where(
        lax.broadcasted_iota(jnp.int32, (_RB, 8), 1) == 0, 1.0, 0.0)

    def fast(_):
        r_i = r_ref[pl.ds(r0, _RB), :]
        rp_i = (_M * jnp.exp(-_LN * c_row)) * r_i
        rp_scr[pl.ds(r0, _RB), 0:_F] = rp_i
        rp_scr[pl.ds(r0, _RB), _F:_F + 8] = ones_pad
        acc_d = lax.dot_general(
            ws, rp_i, (((1,), (0,)), ((), ())),
            preferred_element_type=jnp.float32)
        return (jnp.exp(_LN * c_row) * (acc[:, 0:_F] + acc_d)
                + _M * r_i)

    def slow(_):
        def c_lane_for(l):
            t_chunk = t_lane_ref[pl.ds(l, 1), :]

            def inner(lp, a):
                t_rows = t_sub_ref[pl.ds(lp * _RB, _RB), :]
                m = (t_rows == t_chunk) & jnp.logical_or(
                    lp < l, row_iota < col_iota)
                return a + jnp.sum(m.astype(jnp.float32), axis=0,
                                   keepdims=True)

            return lax.fori_loop(0, l + 1, inner,
                                 jnp.zeros((1, _RB), jnp.float32))

        def wblk(l, acc):
            t_chunk = t_lane_ref[pl.ds(l, 1), :]
            mask = t_row == t_chunk
            d = c_row - c_lane_for(l)
            w = jnp.where(mask, _M * jnp.exp(d * _LN), 0.0)
            rb = r_ref[pl.ds(l * _RB, _RB), :]
            return acc + lax.dot_general(
                w, rb, (((1,), (0,)), ((), ())),
                preferred_element_type=jnp.float32)

        rp_scr[pl.ds(r0, _RB), 0:_F] = jnp.zeros((_RB, _F), jnp.float32)
        rp_scr[pl.ds(r0, _RB), _F:_F + 8] = ones_pad
        a = lax.fori_loop(0, i, wblk, jnp.zeros((_RB, _F), jnp.float32))
        d = c_row - c_lane_for(i)
        mask_diag = eq_diag & (col_iota <= row_iota)
        w = jnp.where(mask_diag, _M * jnp.exp(d * _LN), 0.0)
        a = a + lax.dot_general(
            w, r_ref[pl.ds(r0, _RB), :], (((1,), (0,)), ((), ())),
            preferred_element_type=jnp.float32,
            precision=lax.Precision.HIGHEST)
        return a

    corr_ref[...] = lax.cond(mxc <= _CMAX, fast, slow, 0)
    coef_ref[...] = jnp.exp((c_row + 1.0) * _LN)


def _combine_body(corr_ref, coef_ref, g_ref, o_ref):
    o_ref[...] = corr_ref[...] + coef_ref[...] * g_ref[...]


def _tc_call(t_sub, t_lane, responses, g, interpret=False):
    corr, coef = pl.pallas_call(
        _tc_body,
        grid=(_NB,),
        in_specs=[
            pl.BlockSpec((_B, 1), lambda i: (0, 0)),
            pl.BlockSpec((_NB, _RB), lambda i: (0, 0)),
            pl.BlockSpec((_B, _F), lambda i: (0, 0)),
        ],
        out_specs=[pl.BlockSpec((_RB, _F), lambda i: (i, 0)),
                   pl.BlockSpec((_RB, 1), lambda i: (i, 0))],
        out_shape=[jax.ShapeDtypeStruct((_B, _F), jnp.float32),
                   jax.ShapeDtypeStruct((_B, 1), jnp.float32)],
        scratch_shapes=[pltpu.VMEM((_B, _F + 8), jnp.float32),
                        pltpu.SMEM((1,), jnp.float32)],
        interpret=interpret,
    )(t_sub, t_lane, responses)
    return pl.pallas_call(
        _combine_body,
        grid=(_NB,),
        in_specs=[
            pl.BlockSpec((_RB, _F), lambda i: (i, 0)),
            pl.BlockSpec((_RB, 1), lambda i: (i, 0)),
            pl.BlockSpec((_RB, _F), lambda i: (i, 0)),
        ],
        out_specs=pl.BlockSpec((_RB, _F), lambda i: (i, 0)),
        out_shape=jax.ShapeDtypeStruct((_B, _F), jnp.float32),
        interpret=interpret,
    )(corr, coef, g)


def kernel(responses, targets, mem):
    targets = targets.astype(jnp.int32)
    g = _sc_gather(mem, targets)
    t_sub = targets.reshape(_B, 1)
    t_lane = targets.reshape(_NB, _RB)
    return _tc_call(t_sub, t_lane, responses, g)


# speedup vs baseline: 1.0880x; 1.0880x over previous
import functools
import math

import jax
import jax.numpy as jnp
from jax import lax
from jax.experimental import pallas as pl
from jax.experimental.pallas import tpu as pltpu
from jax.experimental.pallas import tpu_sc as plsc

_M = 0.1
_LN = math.log(1.0 - _M)

_B = 4096
_F = 128
_RB = 1024
_NB = _B // _RB


def _sc_gather(mem, targets):
    info = plsc.get_sparse_core_info()
    nw = info.num_cores * info.num_subcores
    bpw = _B // nw
    mesh = plsc.VectorSubcoreMesh(core_axis_name="c", subcore_axis_name="s")

    @functools.partial(
        pl.kernel,
        mesh=mesh,
        out_type=jax.ShapeDtypeStruct((_B, _F), jnp.float32),
        scratch_types=[
            pltpu.VMEM((bpw,), jnp.int32),
            pltpu.VMEM((bpw, _F), jnp.float32),
            pltpu.SemaphoreType.DMA,
        ],
    )
    def k(table_hbm, idx_hbm, out_hbm, idx_v, rows_v, sem):
        wid = lax.axis_index("s") * info.num_cores + lax.axis_index("c")
        base = wid * bpw
        pltpu.sync_copy(idx_hbm.at[pl.ds(base, bpw)], idx_v)
        pltpu.async_copy(table_hbm.at[idx_v], rows_v, sem).wait()
        pltpu.sync_copy(rows_v, out_hbm.at[pl.ds(base, bpw)])

    return k(mem, targets)


_CMAX = 200.0


def _tc_body(t_sub_ref, t_lane_ref, r_ref, g_ref, o_ref, rp_scr, maxc_ref):
    i = pl.program_id(0)
    r0 = i * _RB
    t_row = t_sub_ref[pl.ds(r0, _RB), :]
    t_col_i = t_lane_ref[pl.ds(i, 1), :]
    row_iota = lax.broadcasted_iota(jnp.int32, (_RB, _RB), 0)
    col_iota = lax.broadcasted_iota(jnp.int32, (_RB, _RB), 1)

    eq_diag = t_row == t_col_i
    mask_strict = eq_diag & (col_iota < row_iota)
    ws = jnp.where(mask_strict, 1.0, 0.0)

    def ablk(l, acc):
        t_chunk = t_lane_ref[pl.ds(l, 1), :]
        w = jnp.where(t_row == t_chunk, 1.0, 0.0)
        return acc + lax.dot_general(
            w, rp_scr[pl.ds(l * _RB, _RB), :], (((1,), (0,)), ((), ())),
            preferred_element_type=jnp.float32)

    acc = lax.fori_loop(0, i, ablk, jnp.zeros((_RB, _F + 8), jnp.float32))
    cntd = lax.dot_general(
        ws, jnp.where(lax.broadcasted_iota(jnp.int32, (_RB, 8), 1) == 0,
                      1.0, 0.0),
        (((1,), (0,)), ((), ())), preferred_element_type=jnp.float32)
    c_row = acc[:, _F:_F + 1] + cntd[:, 0:1]

    mx = jnp.max(c_row)
    mxc = jnp.where(i == 0, mx, jnp.maximum(maxc_ref[0], mx))
    maxc_ref[0] = mxc
    ones_pad = jnp.where(
        lax.broadcasted_iota(jnp.int32, (_RB, 8), 1) == 0, 1.0, 0.0)

    def fast(_):
        r_i = r_ref[pl.ds(r0, _RB), :]
        rp_i = (_M * jnp.exp(-_LN * c_row)) * r_i
        rp_scr[pl.ds(r0, _RB), 0:_F] = rp_i
        rp_scr[pl.ds(r0, _RB), _F:_F + 8] = ones_pad
        acc_d = lax.dot_general(
            ws, rp_i, (((1,), (0,)), ((), ())),
            preferred_element_type=jnp.float32)
        return (jnp.exp(_LN * c_row) * (acc[:, 0:_F] + acc_d)
                + _M * r_i
                + jnp.exp(_LN * (c_row + 1.0)) * g_ref[...])

    def slow(_):
        def c_lane_for(l):
            t_chunk = t_lane_ref[pl.ds(l, 1), :]

            def inner(lp, a):
                t_rows = t_sub_ref[pl.ds(lp * _RB, _RB), :]
                m = (t_rows == t_chunk) & jnp.logical_or(
                    lp < l, row_iota < col_iota)
                return a + jnp.sum(m.astype(jnp.float32), axis=0,
                                   keepdims=True)

            return lax.fori_loop(0, l + 1, inner,
                                 jnp.zeros((1, _RB), jnp.float32))

        def wblk(l, acc):
            t_chunk = t_lane_ref[pl.ds(l, 1), :]
            mask = t_row == t_chunk
            d = c_row - c_lane_for(l)
            w = jnp.where(mask, _M * jnp.exp(d * _LN), 0.0)
            rb = r_ref[pl.ds(l * _RB, _RB), :]
            return acc + lax.dot_general(
                w, rb, (((1,), (0,)), ((), ())),
                preferred_element_type=jnp.float32)

        rp_scr[pl.ds(r0, _RB), 0:_F] = jnp.zeros((_RB, _F), jnp.float32)
        rp_scr[pl.ds(r0, _RB), _F:_F + 8] = ones_pad
        a = lax.fori_loop(0, i, wblk, jnp.zeros((_RB, _F), jnp.float32))
        d = c_row - c_lane_for(i)
        mask_diag = eq_diag & (col_iota <= row_iota)
        w = jnp.where(mask_diag, _M * jnp.exp(d * _LN), 0.0)
        a = a + lax.dot_general(
            w, r_ref[pl.ds(r0, _RB), :], (((1,), (0,)), ((), ())),
            preferred_element_type=jnp.float32,
            precision=lax.Precision.HIGHEST)
        return jnp.exp((c_row + 1.0) * _LN) * g_ref[...] + a

    o_ref[...] = lax.cond(mxc <= _CMAX, fast, slow, 0)


def _tc_call(t_sub, t_lane, responses, g, interpret=False):
    return pl.pallas_call(
        _tc_body,
        grid=(_NB,),
        in_specs=[
            pl.BlockSpec((_B, 1), lambda i: (0, 0)),
            pl.BlockSpec((_NB, _RB), lambda i: (0, 0)),
            pl.BlockSpec((_B, _F), lambda i: (0, 0)),
            pl.BlockSpec((_RB, _F), lambda i: (i, 0)),
        ],
        out_specs=pl.BlockSpec((_RB, _F), lambda i: (i, 0)),
        out_shape=jax.ShapeDtypeStruct((_B, _F), jnp.float32),
        scratch_shapes=[pltpu.VMEM((_B, _F + 8), jnp.float32),
                        pltpu.SMEM((1,), jnp.float32)],
        interpret=interpret,
    )(t_sub, t_lane, responses, g)


def kernel(responses, targets, mem):
    targets = targets.astype(jnp.int32)
    g = _sc_gather(mem, targets)
    t_sub = targets.reshape(_B, 1)
    t_lane = targets.reshape(_NB, _RB)
    return _tc_call(t_sub, t_lane, responses, g)
